# baseline (speedup 1.0000x reference)
import functools

import jax
import jax.numpy as jnp
from jax import lax
from jax.experimental import pallas as pl
from jax.experimental.pallas import tpu as pltpu
from jax.experimental.pallas import tpu_sc as plsc

_E = 64
_BLK = 256
_CHUNK = 128
_NC = 2
_NS = 16
_NW = _NC * _NS


def _router_body(x_ref, wg_ref, p_ref, sched_ref):
    n = x_ref.shape[0]
    e = _E
    t = n // _BLK
    nv = t + e - 1

    logits = lax.dot_general(
        wg_ref[...], x_ref[...],
        dimension_numbers=(((0,), (1,)), ((), ())),
        preferred_element_type=jnp.float32)
    maxv = jnp.max(logits, axis=0, keepdims=True)
    row = lax.broadcasted_iota(jnp.int32, (e, n), 0)
    eid = jnp.min(jnp.where(logits >= maxv, row, e), axis=0, keepdims=True)

    onehot = (eid == lax.broadcasted_iota(jnp.int32, (e, n), 0)).astype(jnp.float32)

    ci = lax.broadcasted_iota(jnp.int32, (_CHUNK, _CHUNK), 0)
    cj = lax.broadcasted_iota(jnp.int32, (_CHUNK, _CHUNK), 1)
    triu = (ci < cj).astype(jnp.float32)
    run = jnp.zeros((e, 1), jnp.float32)
    partial = []
    for c in range(n // _CHUNK):
        oh = onehot[:, c * _CHUNK:(c + 1) * _CHUNK]
        cum_excl = jnp.dot(oh, triu, preferred_element_type=jnp.float32)
        rank_in = jnp.sum(cum_excl * oh, axis=0, keepdims=True)
        base = jnp.sum(run * oh, axis=0, keepdims=True)
        partial.append(rank_in + base)
        run = run + jnp.sum(oh, axis=1, keepdims=True)
    p_partial = jnp.concatenate(partial, axis=1)

    counts = run
    li = lax.broadcasted_iota(jnp.int32, (e, e), 0)
    lj = lax.broadcasted_iota(jnp.int32, (e, e), 1)
    tri_inc = (lj <= li).astype(jnp.float32)
    cum_counts = jnp.dot(tri_inc, counts, precision=lax.Precision.HIGHEST,
                         preferred_element_type=jnp.float32)
    offsets = cum_counts - counts

    offs_tok = jnp.sum(offsets * onehot, axis=0, keepdims=True)
    p_ref[...] = (p_partial + offs_tok).astype(jnp.int32)

    counts_i = counts.astype(jnp.int32)
    offsets_i = offsets.astype(jnp.int32)
    first_blk = offsets_i // _BLK
    last_blk = (offsets_i + counts_i - 1) // _BLK
    span = jnp.where(counts_i > 0, last_blk - first_blk + 1, 0)
    cumspan = jnp.dot(tri_inc, span.astype(jnp.float32),
                      precision=lax.Precision.HIGHEST,
                      preferred_element_type=jnp.float32).astype(jnp.int32)
    total = jnp.sum(span)

    vp = sched_ref.shape[1]
    v_row = lax.broadcasted_iota(jnp.int32, (1, vp), 1)
    valid = v_row < total
    e_raw = jnp.minimum(jnp.sum((cumspan <= v_row).astype(jnp.int32),
                                axis=0, keepdims=True), e - 1)
    e_last = jnp.minimum(jnp.sum((cumspan <= total - 1).astype(jnp.int32)), e - 1)
    e_v = jnp.where(valid, e_raw, e_last)

    sel = (e_v == lax.broadcasted_iota(jnp.int32, (e, vp), 0)).astype(jnp.int32)
    g_cumspan = jnp.sum(sel * cumspan, axis=0, keepdims=True)
    g_span = jnp.sum(sel * span, axis=0, keepdims=True)
    g_first = jnp.sum(sel * first_blk, axis=0, keepdims=True)
    g_off = jnp.sum(sel * offsets_i, axis=0, keepdims=True)
    g_cnt = jnp.sum(sel * counts_i, axis=0, keepdims=True)

    j = v_row - (g_cumspan - g_span)
    b_v = jnp.where(valid, g_first + j, t - 1)
    s_v = jnp.clip(g_off - b_v * _BLK, 0, _BLK)
    n_v = jnp.where(valid, jnp.clip(g_off + g_cnt - b_v * _BLK, 0, _BLK), 0)
    f_v = ((s_v == 0) & valid & (v_row < nv)).astype(jnp.int32)

    sched_ref[...] = jnp.concatenate(
        [e_v, b_v, s_v, n_v, f_v,
         jnp.zeros((3, vp), jnp.int32)], axis=0)


def _router(xf, W_gate):
    n = xf.shape[0]
    return pl.pallas_call(
        _router_body,
        out_shape=[
            jax.ShapeDtypeStruct((1, n), jnp.int32),
            jax.ShapeDtypeStruct((8, 128), jnp.int32),
        ],
    )(xf, W_gate)


def _sc_scatter(xf, pos):
    n, h = xf.shape
    rpw = n // _NW
    mesh = plsc.VectorSubcoreMesh(core_axis_name="c", subcore_axis_name="s")

    @functools.partial(
        pl.kernel,
        mesh=mesh,
        out_type=jax.ShapeDtypeStruct((n, h), jnp.float32),
        scratch_types=[
            pltpu.VMEM((rpw,), jnp.int32),
            pltpu.VMEM((rpw, h), jnp.float32),
            pltpu.SemaphoreType.DMA,
        ],
    )
    def k(x_hbm, idx_hbm, out_hbm, idx_v, rows_v, sem):
        wid = lax.axis_index("s") * _NC + lax.axis_index("c")
        base = wid * rpw
        pltpu.sync_copy(idx_hbm.at[pl.ds(base, rpw)], idx_v)
        pltpu.sync_copy(x_hbm.at[pl.ds(base, rpw)], rows_v)
        pltpu.async_copy(rows_v, out_hbm.at[idx_v], sem).wait()

    return k(xf, pos)


def _sc_gather(ys, pos):
    n, h = ys.shape
    rpw = n // _NW
    mesh = plsc.VectorSubcoreMesh(core_axis_name="c", subcore_axis_name="s")

    @functools.partial(
        pl.kernel,
        mesh=mesh,
        out_type=jax.ShapeDtypeStruct((n, h), jnp.float32),
        scratch_types=[
            pltpu.VMEM((rpw,), jnp.int32),
            pltpu.VMEM((rpw, h), jnp.float32),
            pltpu.SemaphoreType.DMA,
        ],
    )
    def k(y_hbm, idx_hbm, out_hbm, idx_v, rows_v, sem):
        wid = lax.axis_index("s") * _NC + lax.axis_index("c")
        base = wid * rpw
        pltpu.sync_copy(idx_hbm.at[pl.ds(base, rpw)], idx_v)
        pltpu.async_copy(y_hbm.at[idx_v], rows_v, sem).wait()
        pltpu.sync_copy(rows_v, out_hbm.at[pl.ds(base, rpw)])

    return k(ys, pos)


def _ffn_body(sched_ref, pos_ref, x_ref, g_ref, u_ref, d_ref, o_ref):
    v = pl.program_id(0)
    n = x_ref.shape[0]
    b = sched_ref[1, v]
    want = b * _BLK + lax.broadcasted_iota(jnp.int32, (_BLK, n), 0)
    P = (pos_ref[...] == want).astype(jnp.float32)
    xb = jnp.dot(P, x_ref[...], preferred_element_type=jnp.float32)
    g = g_ref[0]
    u = u_ref[0]
    d = d_ref[0]
    hg = jnp.dot(xb, g, preferred_element_type=jnp.float32)
    hu = jnp.dot(xb, u, preferred_element_type=jnp.float32)
    h = (hg * lax.logistic(hg)) * hu
    y = jnp.dot(h, d, preferred_element_type=jnp.float32)
    rid = lax.broadcasted_iota(jnp.int32, y.shape, 0)
    y = jnp.where((rid >= sched_ref[2, v]) & (rid < sched_ref[3, v]), y, 0.0)

    @pl.when(sched_ref[4, v] == 1)
    def _():
        o_ref[...] = y

    @pl.when(sched_ref[4, v] == 0)
    def _():
        o_ref[...] += y


def _grouped_ffn(xf, pos_row, gate_proj, up_proj, down_proj, sched, num_visits):
    n, h = xf.shape
    inter = gate_proj.shape[2]
    grid_spec = pltpu.PrefetchScalarGridSpec(
        num_scalar_prefetch=1,
        grid=(num_visits,),
        in_specs=[
            pl.BlockSpec((1, n), lambda v, S: (0, 0)),
            pl.BlockSpec((n, h), lambda v, S: (0, 0)),
            pl.BlockSpec((1, h, inter), lambda v, S: (S[0, v], 0, 0)),
            pl.BlockSpec((1, h, inter), lambda v, S: (S[0, v], 0, 0)),
            pl.BlockSpec((1, inter, h), lambda v, S: (S[0, v], 0, 0)),
        ],
        out_specs=pl.BlockSpec((_BLK, h), lambda v, S: (S[1, v], 0)),
    )
    return pl.pallas_call(
        _ffn_body,
        grid_spec=grid_spec,
        out_shape=jax.ShapeDtypeStruct((n, h), jnp.float32),
    )(sched, pos_row, xf, gate_proj, up_proj, down_proj)


def kernel(x, W_gate, gate_proj, up_proj, down_proj):
    bsz, seq, h = x.shape
    n = bsz * seq
    nv = n // _BLK + _E - 1
    xf = x.reshape(n, h)

    p_row, sched = _router(xf, W_gate)
    pos = p_row.reshape(n)

    y_sorted = _grouped_ffn(xf, p_row, gate_proj, up_proj, down_proj, sched, nv)
    out = _sc_gather(y_sorted, pos)
    return out.reshape(bsz, seq, h)


# speedup vs baseline: 1.0783x; 1.0783x over previous
import functools

import jax
import jax.numpy as jnp
from jax import lax
from jax.experimental import pallas as pl
from jax.experimental.pallas import tpu as pltpu
from jax.experimental.pallas import tpu_sc as plsc

_E = 64
_BLK = 256
_CHUNK = 128
_NC = 2
_NS = 16
_NW = _NC * _NS


def _router_body(x_ref, wg_ref, p_ref, sched_ref):
    n = x_ref.shape[0]
    e = _E
    t = n // _BLK
    nv = t + e - 1

    logits = lax.dot_general(
        wg_ref[...], x_ref[...],
        dimension_numbers=(((0,), (1,)), ((), ())),
        preferred_element_type=jnp.float32)
    maxv = jnp.max(logits, axis=0, keepdims=True)
    row = lax.broadcasted_iota(jnp.int32, (e, n), 0)
    eid = jnp.min(jnp.where(logits >= maxv, row, e), axis=0, keepdims=True)

    onehot = (eid == lax.broadcasted_iota(jnp.int32, (e, n), 0)).astype(jnp.float32)

    ci = lax.broadcasted_iota(jnp.int32, (_CHUNK, _CHUNK), 0)
    cj = lax.broadcasted_iota(jnp.int32, (_CHUNK, _CHUNK), 1)
    triu = (ci < cj).astype(jnp.float32)
    run = jnp.zeros((e, 1), jnp.float32)
    partial = []
    for c in range(n // _CHUNK):
        oh = onehot[:, c * _CHUNK:(c + 1) * _CHUNK]
        cum_excl = jnp.dot(oh, triu, preferred_element_type=jnp.float32)
        rank_in = jnp.sum(cum_excl * oh, axis=0, keepdims=True)
        base = jnp.sum(run * oh, axis=0, keepdims=True)
        partial.append(rank_in + base)
        run = run + jnp.sum(oh, axis=1, keepdims=True)
    p_partial = jnp.concatenate(partial, axis=1)

    counts = run
    li = lax.broadcasted_iota(jnp.int32, (e, e), 0)
    lj = lax.broadcasted_iota(jnp.int32, (e, e), 1)
    tri_inc = (lj <= li).astype(jnp.float32)
    cum_counts = jnp.dot(tri_inc, counts, precision=lax.Precision.HIGHEST,
                         preferred_element_type=jnp.float32)
    offsets = cum_counts - counts

    offs_tok = jnp.sum(offsets * onehot, axis=0, keepdims=True)
    p_ref[...] = (p_partial + offs_tok).astype(jnp.int32)

    counts_i = counts.astype(jnp.int32)
    offsets_i = offsets.astype(jnp.int32)
    first_blk = offsets_i // _BLK
    last_blk = (offsets_i + counts_i - 1) // _BLK
    span = jnp.where(counts_i > 0, last_blk - first_blk + 1, 0)
    cumspan = jnp.dot(tri_inc, span.astype(jnp.float32),
                      precision=lax.Precision.HIGHEST,
                      preferred_element_type=jnp.float32).astype(jnp.int32)
    total = jnp.sum(span)

    vp = sched_ref.shape[1]
    v_row = lax.broadcasted_iota(jnp.int32, (1, vp), 1)
    valid = v_row < total
    e_raw = jnp.minimum(jnp.sum((cumspan <= v_row).astype(jnp.int32),
                                axis=0, keepdims=True), e - 1)
    e_last = jnp.minimum(jnp.sum((cumspan <= total - 1).astype(jnp.int32)), e - 1)
    e_v = jnp.where(valid, e_raw, e_last)

    sel = (e_v == lax.broadcasted_iota(jnp.int32, (e, vp), 0)).astype(jnp.int32)
    g_cumspan = jnp.sum(sel * cumspan, axis=0, keepdims=True)
    g_span = jnp.sum(sel * span, axis=0, keepdims=True)
    g_first = jnp.sum(sel * first_blk, axis=0, keepdims=True)
    g_off = jnp.sum(sel * offsets_i, axis=0, keepdims=True)
    g_cnt = jnp.sum(sel * counts_i, axis=0, keepdims=True)

    j = v_row - (g_cumspan - g_span)
    b_v = jnp.where(valid, g_first + j, t - 1)
    s_v = jnp.clip(g_off - b_v * _BLK, 0, _BLK)
    n_v = jnp.where(valid, jnp.clip(g_off + g_cnt - b_v * _BLK, 0, _BLK), 0)
    f_v = ((s_v == 0) & valid & (v_row < nv)).astype(jnp.int32)

    sched_ref[...] = jnp.concatenate(
        [e_v, b_v, s_v, n_v, f_v,
         jnp.zeros((3, vp), jnp.int32)], axis=0)


def _router(xf, W_gate):
    n = xf.shape[0]
    return pl.pallas_call(
        _router_body,
        out_shape=[
            jax.ShapeDtypeStruct((1, n), jnp.int32),
            jax.ShapeDtypeStruct((8, 128), jnp.int32),
        ],
    )(xf, W_gate)


def _sc_scatter(xf, pos):
    n, h = xf.shape
    rpw = n // _NW
    mesh = plsc.VectorSubcoreMesh(core_axis_name="c", subcore_axis_name="s")

    @functools.partial(
        pl.kernel,
        mesh=mesh,
        out_type=jax.ShapeDtypeStruct((n, h), jnp.float32),
        scratch_types=[
            pltpu.VMEM((rpw,), jnp.int32),
            pltpu.VMEM((rpw, h), jnp.float32),
            pltpu.SemaphoreType.DMA,
        ],
    )
    def k(x_hbm, idx_hbm, out_hbm, idx_v, rows_v, sem):
        wid = lax.axis_index("s") * _NC + lax.axis_index("c")
        base = wid * rpw
        pltpu.sync_copy(idx_hbm.at[pl.ds(base, rpw)], idx_v)
        pltpu.sync_copy(x_hbm.at[pl.ds(base, rpw)], rows_v)
        pltpu.async_copy(rows_v, out_hbm.at[idx_v], sem).wait()

    return k(xf, pos)


def _sc_gather(ys, pos):
    n, h = ys.shape
    rpw = n // _NW
    mesh = plsc.VectorSubcoreMesh(core_axis_name="c", subcore_axis_name="s")

    @functools.partial(
        pl.kernel,
        mesh=mesh,
        out_type=jax.ShapeDtypeStruct((n, h), jnp.float32),
        scratch_types=[
            pltpu.VMEM((rpw,), jnp.int32),
            pltpu.VMEM((rpw, h), jnp.float32),
            pltpu.SemaphoreType.DMA,
        ],
    )
    def k(y_hbm, idx_hbm, out_hbm, idx_v, rows_v, sem):
        wid = lax.axis_index("s") * _NC + lax.axis_index("c")
        base = wid * rpw
        pltpu.sync_copy(idx_hbm.at[pl.ds(base, rpw)], idx_v)
        pltpu.async_copy(y_hbm.at[idx_v], rows_v, sem).wait()
        pltpu.sync_copy(rows_v, out_hbm.at[pl.ds(base, rpw)])

    return k(ys, pos)


def _ffn_body(sched_ref, x_ref, g_ref, u_ref, d_ref, o_ref):
    v = pl.program_id(0)
    b = sched_ref[1, v]
    xb = x_ref[pl.ds(b * _BLK, _BLK), :]
    g = g_ref[0]
    u = u_ref[0]
    d = d_ref[0]
    hg = jnp.dot(xb, g, preferred_element_type=jnp.float32)
    hu = jnp.dot(xb, u, preferred_element_type=jnp.float32)
    h = (hg * lax.logistic(hg)) * hu
    y = jnp.dot(h, d, preferred_element_type=jnp.float32)
    rid = lax.broadcasted_iota(jnp.int32, y.shape, 0)
    y = jnp.where((rid >= sched_ref[2, v]) & (rid < sched_ref[3, v]), y, 0.0)

    @pl.when(sched_ref[4, v] == 1)
    def _():
        o_ref[pl.ds(b * _BLK, _BLK), :] = y

    @pl.when(sched_ref[4, v] == 0)
    def _():
        o_ref[pl.ds(b * _BLK, _BLK), :] += y


def _grouped_ffn(x_sorted, gate_proj, up_proj, down_proj, sched, num_visits):
    n, h = x_sorted.shape
    inter = gate_proj.shape[2]
    grid_spec = pltpu.PrefetchScalarGridSpec(
        num_scalar_prefetch=1,
        grid=(num_visits,),
        in_specs=[
            pl.BlockSpec((n, h), lambda v, S: (0, 0)),
            pl.BlockSpec((1, h, inter), lambda v, S: (S[0, v], 0, 0)),
            pl.BlockSpec((1, h, inter), lambda v, S: (S[0, v], 0, 0)),
            pl.BlockSpec((1, inter, h), lambda v, S: (S[0, v], 0, 0)),
        ],
        out_specs=pl.BlockSpec((n, h), lambda v, S: (0, 0)),
    )
    return pl.pallas_call(
        _ffn_body,
        grid_spec=grid_spec,
        out_shape=jax.ShapeDtypeStruct((n, h), jnp.float32),
    )(sched, x_sorted, gate_proj, up_proj, down_proj)


def kernel(x, W_gate, gate_proj, up_proj, down_proj):
    bsz, seq, h = x.shape
    n = bsz * seq
    nv = n // _BLK + _E - 1
    xf = x.reshape(n, h)

    p_row, sched = _router(xf, W_gate)
    pos = p_row.reshape(n)

    x_sorted = _sc_scatter(xf, pos)
    y_sorted = _grouped_ffn(x_sorted, gate_proj, up_proj, down_proj, sched, nv)
    out = _sc_gather(y_sorted, pos)
    return out.reshape(bsz, seq, h)


# speedup vs baseline: 1.0913x; 1.0120x over previous
import functools

import jax
import jax.numpy as jnp
from jax import lax
from jax.experimental import pallas as pl
from jax.experimental.pallas import tpu as pltpu
from jax.experimental.pallas import tpu_sc as plsc

_E = 64
_BLK = 256
_CHUNK = 128
_NC = 2
_NS = 16
_NW = _NC * _NS


def _router_body(x_ref, wg_ref, p_ref, sched_ref):
    n = x_ref.shape[0]
    e = _E
    t = n // _BLK
    nv = t + e - 1

    logits = lax.dot_general(
        wg_ref[...], x_ref[...],
        dimension_numbers=(((0,), (1,)), ((), ())),
        preferred_element_type=jnp.float32)
    maxv = jnp.max(logits, axis=0, keepdims=True)
    row = lax.broadcasted_iota(jnp.int32, (e, n), 0)
    eid = jnp.min(jnp.where(logits >= maxv, row, e), axis=0, keepdims=True)

    onehot = (eid == lax.broadcasted_iota(jnp.int32, (e, n), 0)).astype(jnp.float32)

    ci = lax.broadcasted_iota(jnp.int32, (_CHUNK, _CHUNK), 0)
    cj = lax.broadcasted_iota(jnp.int32, (_CHUNK, _CHUNK), 1)
    triu = (ci < cj).astype(jnp.float32)
    run = jnp.zeros((e, 1), jnp.float32)
    partial = []
    for c in range(n // _CHUNK):
        oh = onehot[:, c * _CHUNK:(c + 1) * _CHUNK]
        cum_excl = jnp.dot(oh, triu, preferred_element_type=jnp.float32)
        rank_in = jnp.sum(cum_excl * oh, axis=0, keepdims=True)
        base = jnp.sum(run * oh, axis=0, keepdims=True)
        partial.append(rank_in + base)
        run = run + jnp.sum(oh, axis=1, keepdims=True)
    p_partial = jnp.concatenate(partial, axis=1)

    counts = run
    li = lax.broadcasted_iota(jnp.int32, (e, e), 0)
    lj = lax.broadcasted_iota(jnp.int32, (e, e), 1)
    tri_inc = (lj <= li).astype(jnp.float32)
    cum_counts = jnp.dot(tri_inc, counts, precision=lax.Precision.HIGHEST,
                         preferred_element_type=jnp.float32)
    offsets = cum_counts - counts

    offs_tok = jnp.sum(offsets * onehot, axis=0, keepdims=True)
    p_ref[...] = (p_partial + offs_tok).astype(jnp.int32)

    counts_i = counts.astype(jnp.int32)
    offsets_i = offsets.astype(jnp.int32)
    first_blk = offsets_i // _BLK
    last_blk = (offsets_i + counts_i - 1) // _BLK
    span = jnp.where(counts_i > 0, last_blk - first_blk + 1, 0)
    cumspan = jnp.dot(tri_inc, span.astype(jnp.float32),
                      precision=lax.Precision.HIGHEST,
                      preferred_element_type=jnp.float32).astype(jnp.int32)
    total = jnp.sum(span)

    vp = sched_ref.shape[1]
    v_row = lax.broadcasted_iota(jnp.int32, (1, vp), 1)
    valid = v_row < total
    e_raw = jnp.minimum(jnp.sum((cumspan <= v_row).astype(jnp.int32),
                                axis=0, keepdims=True), e - 1)
    e_last = jnp.minimum(jnp.sum((cumspan <= total - 1).astype(jnp.int32)), e - 1)
    e_v = jnp.where(valid, e_raw, e_last)

    sel = (e_v == lax.broadcasted_iota(jnp.int32, (e, vp), 0)).astype(jnp.int32)
    g_cumspan = jnp.sum(sel * cumspan, axis=0, keepdims=True)
    g_span = jnp.sum(sel * span, axis=0, keepdims=True)
    g_first = jnp.sum(sel * first_blk, axis=0, keepdims=True)
    g_off = jnp.sum(sel * offsets_i, axis=0, keepdims=True)
    g_cnt = jnp.sum(sel * counts_i, axis=0, keepdims=True)

    j = v_row - (g_cumspan - g_span)
    b_v = jnp.where(valid, g_first + j, t - 1)
    s_v = jnp.clip(g_off - b_v * _BLK, 0, _BLK)
    n_v = jnp.where(valid, jnp.clip(g_off + g_cnt - b_v * _BLK, 0, _BLK), 0)
    f_v = ((s_v == 0) & valid & (v_row < nv)).astype(jnp.int32)

    sched_ref[...] = jnp.concatenate(
        [e_v, b_v, s_v, n_v, f_v,
         jnp.zeros((3, vp), jnp.int32)], axis=0)


def _router(xf, W_gate):
    n = xf.shape[0]
    return pl.pallas_call(
        _router_body,
        out_shape=[
            jax.ShapeDtypeStruct((1, n), jnp.int32),
            jax.ShapeDtypeStruct((8, 128), jnp.int32),
        ],
    )(xf, W_gate)


def _sc_scatter(xf, pos):
    n, h = xf.shape
    rpw = n // _NW
    mesh = plsc.VectorSubcoreMesh(core_axis_name="c", subcore_axis_name="s")

    @functools.partial(
        pl.kernel,
        mesh=mesh,
        out_type=jax.ShapeDtypeStruct((n, h), jnp.float32),
        scratch_types=[
            pltpu.VMEM((rpw,), jnp.int32),
            pltpu.VMEM((rpw, h), jnp.float32),
            pltpu.SemaphoreType.DMA,
        ],
    )
    def k(x_hbm, idx_hbm, out_hbm, idx_v, rows_v, sem):
        wid = lax.axis_index("s") * _NC + lax.axis_index("c")
        base = wid * rpw
        pltpu.sync_copy(idx_hbm.at[pl.ds(base, rpw)], idx_v)
        pltpu.sync_copy(x_hbm.at[pl.ds(base, rpw)], rows_v)
        pltpu.async_copy(rows_v, out_hbm.at[idx_v], sem).wait()

    return k(xf, pos)


def _sc_gather(ys, pos):
    n, h = ys.shape
    rpw = n // _NW
    mesh = plsc.VectorSubcoreMesh(core_axis_name="c", subcore_axis_name="s")

    @functools.partial(
        pl.kernel,
        mesh=mesh,
        out_type=jax.ShapeDtypeStruct((n, h), jnp.float32),
        scratch_types=[
            pltpu.VMEM((rpw,), jnp.int32),
            pltpu.VMEM((rpw, h), jnp.float32),
            pltpu.SemaphoreType.DMA,
        ],
    )
    def k(y_hbm, idx_hbm, out_hbm, idx_v, rows_v, sem):
        wid = lax.axis_index("s") * _NC + lax.axis_index("c")
        base = wid * rpw
        pltpu.sync_copy(idx_hbm.at[pl.ds(base, rpw)], idx_v)
        pltpu.async_copy(y_hbm.at[idx_v], rows_v, sem).wait()
        pltpu.sync_copy(rows_v, out_hbm.at[pl.ds(base, rpw)])

    return k(ys, pos)


def _ffn_body(sched_ref, x_ref, g_ref, u_ref, d_ref, o_ref):
    v = pl.program_id(0)
    xb = x_ref[...]
    g = g_ref[0]
    u = u_ref[0]
    d = d_ref[0]
    hg = jnp.dot(xb, g, preferred_element_type=jnp.float32,
                 precision=lax.Precision.DEFAULT)
    hu = jnp.dot(xb, u, preferred_element_type=jnp.float32,
                 precision=lax.Precision.DEFAULT)
    h = (hg * lax.logistic(hg)) * hu
    y = jnp.dot(h, d, preferred_element_type=jnp.float32,
                precision=lax.Precision.DEFAULT)
    rid = lax.broadcasted_iota(jnp.int32, y.shape, 0)
    y = jnp.where((rid >= sched_ref[2, v]) & (rid < sched_ref[3, v]), y, 0.0)

    @pl.when(sched_ref[4, v] == 1)
    def _():
        o_ref[...] = y

    @pl.when(sched_ref[4, v] == 0)
    def _():
        o_ref[...] += y


def _grouped_ffn(x_sorted, gate_proj, up_proj, down_proj, sched, num_visits):
    n, h = x_sorted.shape
    inter = gate_proj.shape[2]
    grid_spec = pltpu.PrefetchScalarGridSpec(
        num_scalar_prefetch=1,
        grid=(num_visits,),
        in_specs=[
            pl.BlockSpec((_BLK, h), lambda v, S: (S[1, v], 0)),
            pl.BlockSpec((1, h, inter), lambda v, S: (S[0, v], 0, 0)),
            pl.BlockSpec((1, h, inter), lambda v, S: (S[0, v], 0, 0)),
            pl.BlockSpec((1, inter, h), lambda v, S: (S[0, v], 0, 0)),
        ],
        out_specs=pl.BlockSpec((_BLK, h), lambda v, S: (S[1, v], 0)),
    )
    return pl.pallas_call(
        _ffn_body,
        grid_spec=grid_spec,
        out_shape=jax.ShapeDtypeStruct((n, h), jnp.float32),
    )(sched, x_sorted, gate_proj, up_proj, down_proj)


def kernel(x, W_gate, gate_proj, up_proj, down_proj):
    bsz, seq, h = x.shape
    n = bsz * seq
    nv = n // _BLK + _E - 1
    xf = x.reshape(n, h)

    p_row, sched = _router(xf, W_gate)
    pos = p_row.reshape(n)

    x_sorted = _sc_scatter(xf, pos)
    y_sorted = _grouped_ffn(x_sorted, gate_proj, up_proj, down_proj, sched, nv)
    out = _sc_gather(y_sorted, pos)
    return out.reshape(bsz, seq, h)
